# Initial kernel scaffold; baseline (speedup 1.0000x reference)
import jax
import jax.numpy as jnp
from jax.experimental import pallas as pl


def kernel(x, edge_indices, gcn1_W, gcn1_b, gcn2_W, gcn2_b, W_ih, W_hh, b_ih, b_hh, fc_W, fc_b):
    raise NotImplementedError("write your pallas kernel here")


# speedup vs baseline: 3.9912x; 3.9912x over previous
import functools

import jax
import jax.numpy as jnp
from jax import lax
from jax.experimental import pallas as pl
from jax.experimental.pallas import tpu as pltpu
from jax.experimental.pallas import tpu_sc as plsc

N = 10000
T = 10
D_IN = 128
HID = 256
OUT_DIM = 128
HALF = 128
E = 160000

NC = 2
NS = 16
CHUNK = 128
NCHUNK = 80
EPT = NCHUNK * CHUNK
E_PAD = EPT * NS
NROW = 10240
ZPT = NROW // NS
RPT = 624
TAIL0 = RPT * NS
TAIL = N - TAIL0

BN = 1000
NB = N // BN

_mesh = plsc.VectorSubcoreMesh(
    core_axis_name="c", subcore_axis_name="s", num_cores=NC, num_subcores=NS)


@functools.partial(
    pl.kernel,
    out_type=jax.ShapeDtypeStruct((T, NC, N, HALF), jnp.float32),
    mesh=_mesh,
    scratch_types=[
        pltpu.VMEM_SHARED((NROW, HALF), jnp.float32),
        pltpu.VMEM((NCHUNK // 8, 8, CHUNK), jnp.int32),
        pltpu.VMEM((NCHUNK // 8, 8, CHUNK), jnp.int32),
        pltpu.VMEM((CHUNK, HALF), jnp.float32),
        pltpu.SemaphoreType.DMA,
    ],
)
def _edge_kernel(y_hbm, src_hbm, dst_hbm, out_hbm, acc, src_v, dst_v, gbuf, sem):
    c = lax.axis_index("c")
    s = lax.axis_index("s")

    def per_t(t, carry):
        pltpu.sync_copy(y_hbm.at[t, c, pl.ds(s * RPT, RPT)],
                        acc.at[pl.ds(s * RPT, RPT)])

        @pl.when(s == NS - 1)
        def _():
            pltpu.sync_copy(y_hbm.at[t, c, pl.ds(TAIL0, TAIL)],
                            acc.at[pl.ds(TAIL0, TAIL)])

        pltpu.sync_copy(src_hbm.at[t, s], src_v)
        pltpu.sync_copy(dst_hbm.at[t, s], dst_v)
        plsc.subcore_barrier()

        def per_group(g, carry2):
            for r in range(8):
                pltpu.async_copy(y_hbm.at[t, c].at[src_v.at[g, r]], gbuf,
                                 sem).wait()
                pltpu.sync_copy(gbuf, acc.at[dst_v.at[g, r]], add=True)
            return carry2

        lax.fori_loop(0, NCHUNK // 8, per_group, 0)
        plsc.subcore_barrier()
        pltpu.sync_copy(acc.at[pl.ds(s * RPT, RPT)],
                        out_hbm.at[t, c, pl.ds(s * RPT, RPT)])

        @pl.when(s == NS - 1)
        def _():
            pltpu.sync_copy(acc.at[pl.ds(TAIL0, TAIL)],
                            out_hbm.at[t, c, pl.ds(TAIL0, TAIL)])

        plsc.subcore_barrier()
        return carry

    lax.fori_loop(0, T, per_t, 0)


def _tc1_body(x_ref, deg_ref, w_ref, o_ref):
    xt = x_ref[0]
    y = jnp.dot(xt, w_ref[...], preferred_element_type=jnp.float32)
    dinv = lax.rsqrt(deg_ref[0, 0, 0])
    y = y * dinv[:, None]
    o_ref[0, 0] = y[:, :HALF]
    o_ref[0, 1] = y[:, HALF:]


def _tc2_body(z_ref, deg_ref, b1_ref, w2_ref, o_ref):
    z = jnp.concatenate([z_ref[0, 0], z_ref[0, 1]], axis=1)
    dinv = lax.rsqrt(deg_ref[0, 0, 0])
    h = jnp.maximum(z * dinv[:, None] + b1_ref[...], 0.0)
    y = jnp.dot(h, w2_ref[...], preferred_element_type=jnp.float32)
    y = y * dinv[:, None]
    o_ref[0, 0] = y[:, :HALF]
    o_ref[0, 1] = y[:, HALF:]


def _lstm_body(z_ref, deg_ref, b2_ref, wih_ref, whh_ref, bih_ref, bhh_ref,
               fcw_ref, fcb_ref, o_ref):
    h = jnp.zeros((BN, HID), jnp.float32)
    cc = jnp.zeros((BN, HID), jnp.float32)
    bg = bih_ref[...] + bhh_ref[...]
    for t in range(T):
        z = jnp.concatenate([z_ref[t, 0], z_ref[t, 1]], axis=1)
        dinv = lax.rsqrt(deg_ref[t, 0, 0])
        xt = z * dinv[:, None] + b2_ref[...]
        g = (jnp.dot(xt, wih_ref[...], preferred_element_type=jnp.float32)
             + jnp.dot(h, whh_ref[...], preferred_element_type=jnp.float32)
             + bg)
        gi = jax.nn.sigmoid(g[:, :HID])
        gf = jax.nn.sigmoid(g[:, HID:2 * HID])
        gg = jnp.tanh(g[:, 2 * HID:3 * HID])
        go = jax.nn.sigmoid(g[:, 3 * HID:])
        cc = gf * cc + gi * gg
        h = go * jnp.tanh(cc)
    o_ref[...] = (jnp.dot(h, fcw_ref[...], preferred_element_type=jnp.float32)
                  + fcb_ref[...])


def _tc1_call(xr, deg3, w1):
    return pl.pallas_call(
        _tc1_body,
        grid=(T, NB),
        in_specs=[
            pl.BlockSpec((1, BN, D_IN), lambda t, i: (t, i, 0)),
            pl.BlockSpec((1, 1, 1, BN), lambda t, i: (t, i, 0, 0)),
            pl.BlockSpec((D_IN, HID), lambda t, i: (0, 0)),
        ],
        out_specs=pl.BlockSpec((1, NC, BN, HALF), lambda t, i: (t, 0, i, 0)),
        out_shape=jax.ShapeDtypeStruct((T, NC, N, HALF), jnp.float32),
    )(xr, deg3, w1)


def _tc2_call(z1, deg3, b1, w2):
    return pl.pallas_call(
        _tc2_body,
        grid=(T, NB),
        in_specs=[
            pl.BlockSpec((1, NC, BN, HALF), lambda t, i: (t, 0, i, 0)),
            pl.BlockSpec((1, 1, 1, BN), lambda t, i: (t, i, 0, 0)),
            pl.BlockSpec((1, HID), lambda t, i: (0, 0)),
            pl.BlockSpec((HID, HID), lambda t, i: (0, 0)),
        ],
        out_specs=pl.BlockSpec((1, NC, BN, HALF), lambda t, i: (t, 0, i, 0)),
        out_shape=jax.ShapeDtypeStruct((T, NC, N, HALF), jnp.float32),
    )(z1, deg3, b1, w2)


def _lstm_call(z2, deg3, b2, wihT, whhT, bih, bhh, fcw, fcb):
    return pl.pallas_call(
        _lstm_body,
        grid=(NB,),
        in_specs=[
            pl.BlockSpec((T, NC, BN, HALF), lambda i: (0, 0, i, 0)),
            pl.BlockSpec((T, 1, 1, BN), lambda i: (0, i, 0, 0)),
            pl.BlockSpec((1, HID), lambda i: (0, 0)),
            pl.BlockSpec((HID, 4 * HID), lambda i: (0, 0)),
            pl.BlockSpec((HID, 4 * HID), lambda i: (0, 0)),
            pl.BlockSpec((1, 4 * HID), lambda i: (0, 0)),
            pl.BlockSpec((1, 4 * HID), lambda i: (0, 0)),
            pl.BlockSpec((HID, OUT_DIM), lambda i: (0, 0)),
            pl.BlockSpec((1, OUT_DIM), lambda i: (0, 0)),
        ],
        out_specs=pl.BlockSpec((BN, OUT_DIM), lambda i: (i, 0)),
        out_shape=jax.ShapeDtypeStruct((N, OUT_DIM), jnp.float32),
    )(z2, deg3, b2, wihT, whhT, bih, bhh, fcw, fcb)


@jax.jit
def _run(x, edge_indices, gcn1_W, gcn1_b, gcn2_W, gcn2_b, W_ih, W_hh,
         b_ih, b_hh, fc_W, fc_b):
    ei = edge_indices.astype(jnp.int32)
    pad = E_PAD - E
    src = jnp.pad(ei[:, 0], ((0, 0), (0, pad)))
    dst = jnp.pad(ei[:, 1], ((0, 0), (0, pad)), constant_values=N)
    src_r = src.reshape(T, NS, NCHUNK // 8, 8, CHUNK)
    dst_r = dst.reshape(T, NS, NCHUNK // 8, 8, CHUNK)

    onesy = jnp.ones((T, NC, N, HALF), jnp.float32)
    degp1 = _edge_kernel(onesy, src_r, dst_r)
    deg3 = degp1[:, 0, :, 0].reshape(T, NB, 1, BN)

    xr = jnp.transpose(x[0], (1, 0, 2))
    y1 = _tc1_call(xr, deg3, gcn1_W)
    z1 = _edge_kernel(y1, src_r, dst_r)
    y2 = _tc2_call(z1, deg3, gcn1_b.reshape(1, HID), gcn2_W)
    z2 = _edge_kernel(y2, src_r, dst_r)
    out = _lstm_call(z2, deg3, gcn2_b.reshape(1, HID), W_ih.T, W_hh.T,
                     b_ih.reshape(1, 4 * HID), b_hh.reshape(1, 4 * HID),
                     fc_W, fc_b.reshape(1, OUT_DIM))
    return out.reshape(1, N, OUT_DIM)


def kernel(x, edge_indices, gcn1_W, gcn1_b, gcn2_W, gcn2_b, W_ih, W_hh,
           b_ih, b_hh, fc_W, fc_b):
    return _run(x, edge_indices, gcn1_W, gcn1_b, gcn2_W, gcn2_b, W_ih, W_hh,
                b_ih, b_hh, fc_W, fc_b)


# speedup vs baseline: 6.1653x; 1.5447x over previous
import functools

import jax
import jax.numpy as jnp
from jax import lax
from jax.experimental import pallas as pl
from jax.experimental.pallas import tpu as pltpu
from jax.experimental.pallas import tpu_sc as plsc

N = 10000
T = 10
D_IN = 128
HID = 256
OUT_DIM = 128
HALF = 128
E = 160000

NC = 2
NS = 16
CHUNK = 128
NCHUNK = 80
EPT = NCHUNK * CHUNK
E_PAD = EPT * NS
NROW = 10240
ZPT = NROW // NS
RPT = 624
TAIL0 = RPT * NS
TAIL = N - TAIL0

SROW = 128
NG = 10
HG = 5

BN = 1000
NB = N // BN

T_PER_SC = T // NC

_mesh = plsc.VectorSubcoreMesh(
    core_axis_name="c", subcore_axis_name="s", num_cores=NC, num_subcores=NS)


@functools.partial(
    pl.kernel,
    out_type=jax.ShapeDtypeStruct((T, N, HALF), jnp.float32),
    mesh=_mesh,
    scratch_types=[
        pltpu.VMEM_SHARED((NROW, HALF), jnp.float32),
        pltpu.VMEM((NG, 8, CHUNK), jnp.int32),
        pltpu.VMEM((SROW, HALF), jnp.float32),
        pltpu.SemaphoreType.DMA,
    ],
)
def _deg_kernel(dst_hbm, ones_hbm, out_hbm, acc, dst_v, ones_v, sem):
    c = lax.axis_index("c")
    s = lax.axis_index("s")
    pltpu.sync_copy(ones_hbm.at[pl.ds(0, SROW)], ones_v)

    def per_t(i, carry):
        t = c * T_PER_SC + i
        pltpu.sync_copy(ones_hbm, acc.at[pl.ds(s * ZPT, ZPT)])
        pltpu.sync_copy(dst_hbm.at[t, s], dst_v)
        plsc.subcore_barrier()

        def fire(g, carry2):
            for r in range(8):
                pltpu.async_copy(ones_v, acc.at[dst_v.at[g, r]], sem,
                                 add=True)
            return carry2

        lax.fori_loop(0, NG, fire, 0)

        def drain(g, carry2):
            for r in range(8):
                pltpu.make_async_copy(
                    ones_v, acc.at[dst_v.at[g, r]], sem).wait()
            return carry2

        lax.fori_loop(0, NG, drain, 0)
        plsc.subcore_barrier()
        pltpu.sync_copy(acc.at[pl.ds(s * RPT, RPT)],
                        out_hbm.at[t, pl.ds(s * RPT, RPT)])

        @pl.when(s == NS - 1)
        def _():
            pltpu.sync_copy(acc.at[pl.ds(TAIL0, TAIL)],
                            out_hbm.at[t, pl.ds(TAIL0, TAIL)])

        plsc.subcore_barrier()
        return carry

    lax.fori_loop(0, T_PER_SC, per_t, 0)


@functools.partial(
    pl.kernel,
    out_type=jax.ShapeDtypeStruct((T, NC, N, HALF), jnp.float32),
    mesh=_mesh,
    scratch_types=[
        pltpu.VMEM_SHARED((NROW, HALF), jnp.float32),
        pltpu.VMEM((HG, 8, CHUNK), jnp.int32),
        pltpu.VMEM((HG, 8, CHUNK), jnp.int32),
        pltpu.VMEM((SROW, HALF), jnp.float32),
        pltpu.VMEM((SROW, HALF), jnp.float32),
        pltpu.SemaphoreType.DMA,
        pltpu.SemaphoreType.DMA,
        pltpu.SemaphoreType.DMA,
        pltpu.SemaphoreType.DMA,
    ],
)
def _edge_kernel(y_hbm, src_hbm, dst_hbm, out_hbm, acc, src_v, dst_v,
                 gbufA, gbufB, gsemA, gsemB, ssemA, ssemB):
    c = lax.axis_index("c")
    s = lax.axis_index("s")

    def per_t(t, carry):
        yt = y_hbm.at[t, c]

        def gslice(g, r):
            return yt.at[src_v.at[g, r]]

        def sslice(g, r):
            return acc.at[dst_v.at[g, r]]

        pltpu.sync_copy(y_hbm.at[t, c, pl.ds(s * RPT, RPT)],
                        acc.at[pl.ds(s * RPT, RPT)])

        @pl.when(s == NS - 1)
        def _():
            pltpu.sync_copy(y_hbm.at[t, c, pl.ds(TAIL0, TAIL)],
                            acc.at[pl.ds(TAIL0, TAIL)])

        plsc.subcore_barrier()

        def per_half(h, carry2):
            pltpu.sync_copy(src_hbm.at[t, s, pl.ds(h * HG, HG)], src_v)
            pltpu.sync_copy(dst_hbm.at[t, s, pl.ds(h * HG, HG)], dst_v)
            pltpu.async_copy(gslice(0, 0), gbufA, gsemA)
            pltpu.async_copy(y_hbm.at[t, c, pl.ds(0, SROW)], gbufB, ssemB)

            def per_group(g, carry3):
                for r in range(8):
                    if r % 2 == 0:
                        buf, gsem, ssem = gbufA, gsemA, ssemA
                        obuf, ogsem, ossem = gbufB, gsemB, ssemB
                    else:
                        buf, gsem, ssem = gbufB, gsemB, ssemB
                        obuf, ogsem, ossem = gbufA, gsemA, ssemA
                    pltpu.make_async_copy(gslice(g, r), buf, gsem).wait()
                    pltpu.async_copy(buf, sslice(g, r), ssem, add=True)
                    pltpu.make_async_copy(y_hbm.at[t, c, pl.ds(0, SROW)],
                                          obuf, ossem).wait()
                    if r == 7:
                        gn = jnp.minimum(g + 1, HG - 1)
                        pltpu.async_copy(gslice(gn, 0), obuf, ogsem)
                    else:
                        pltpu.async_copy(gslice(g, r + 1), obuf, ogsem)
                return carry3

            lax.fori_loop(0, HG, per_group, 0)
            pltpu.make_async_copy(gslice(0, 0), gbufA, gsemA).wait()
            pltpu.make_async_copy(y_hbm.at[t, c, pl.ds(0, SROW)], gbufB,
                                  ssemB).wait()
            return carry2

        lax.fori_loop(0, 2, per_half, 0)
        plsc.subcore_barrier()
        pltpu.sync_copy(acc.at[pl.ds(s * RPT, RPT)],
                        out_hbm.at[t, c, pl.ds(s * RPT, RPT)])

        @pl.when(s == NS - 1)
        def _():
            pltpu.sync_copy(acc.at[pl.ds(TAIL0, TAIL)],
                            out_hbm.at[t, c, pl.ds(TAIL0, TAIL)])

        plsc.subcore_barrier()
        return carry

    lax.fori_loop(0, T, per_t, 0)


def _tc1_body(x_ref, deg_ref, w_ref, o_ref):
    xt = x_ref[0]
    y = jnp.dot(xt, w_ref[...], preferred_element_type=jnp.float32)
    dinv = lax.rsqrt(deg_ref[0, 0, 0])
    y = y * dinv[:, None]
    o_ref[0, 0] = y[:, :HALF]
    o_ref[0, 1] = y[:, HALF:]


def _tc2_body(z_ref, deg_ref, b1_ref, w2_ref, o_ref):
    z = jnp.concatenate([z_ref[0, 0], z_ref[0, 1]], axis=1)
    dinv = lax.rsqrt(deg_ref[0, 0, 0])
    h = jnp.maximum(z * dinv[:, None] + b1_ref[...], 0.0)
    y = jnp.dot(h, w2_ref[...], preferred_element_type=jnp.float32)
    y = y * dinv[:, None]
    o_ref[0, 0] = y[:, :HALF]
    o_ref[0, 1] = y[:, HALF:]


def _lstm_body(z_ref, deg_ref, b2_ref, wih_ref, whh_ref, bih_ref, bhh_ref,
               fcw_ref, fcb_ref, o_ref):
    h = jnp.zeros((BN, HID), jnp.float32)
    cc = jnp.zeros((BN, HID), jnp.float32)
    bg = bih_ref[...] + bhh_ref[...]
    for t in range(T):
        z = jnp.concatenate([z_ref[t, 0], z_ref[t, 1]], axis=1)
        dinv = lax.rsqrt(deg_ref[t, 0, 0])
        xt = z * dinv[:, None] + b2_ref[...]
        g = (jnp.dot(xt, wih_ref[...], preferred_element_type=jnp.float32)
             + jnp.dot(h, whh_ref[...], preferred_element_type=jnp.float32)
             + bg)
        gi = jax.nn.sigmoid(g[:, :HID])
        gf = jax.nn.sigmoid(g[:, HID:2 * HID])
        gg = jnp.tanh(g[:, 2 * HID:3 * HID])
        go = jax.nn.sigmoid(g[:, 3 * HID:])
        cc = gf * cc + gi * gg
        h = go * jnp.tanh(cc)
    o_ref[...] = (jnp.dot(h, fcw_ref[...], preferred_element_type=jnp.float32)
                  + fcb_ref[...])


def _tc1_call(xr, deg3, w1):
    return pl.pallas_call(
        _tc1_body,
        grid=(T, NB),
        in_specs=[
            pl.BlockSpec((1, BN, D_IN), lambda t, i: (t, i, 0)),
            pl.BlockSpec((1, 1, 1, BN), lambda t, i: (t, i, 0, 0)),
            pl.BlockSpec((D_IN, HID), lambda t, i: (0, 0)),
        ],
        out_specs=pl.BlockSpec((1, NC, BN, HALF), lambda t, i: (t, 0, i, 0)),
        out_shape=jax.ShapeDtypeStruct((T, NC, N, HALF), jnp.float32),
    )(xr, deg3, w1)


def _tc2_call(z1, deg3, b1, w2):
    return pl.pallas_call(
        _tc2_body,
        grid=(T, NB),
        in_specs=[
            pl.BlockSpec((1, NC, BN, HALF), lambda t, i: (t, 0, i, 0)),
            pl.BlockSpec((1, 1, 1, BN), lambda t, i: (t, i, 0, 0)),
            pl.BlockSpec((1, HID), lambda t, i: (0, 0)),
            pl.BlockSpec((HID, HID), lambda t, i: (0, 0)),
        ],
        out_specs=pl.BlockSpec((1, NC, BN, HALF), lambda t, i: (t, 0, i, 0)),
        out_shape=jax.ShapeDtypeStruct((T, NC, N, HALF), jnp.float32),
    )(z1, deg3, b1, w2)


def _lstm_call(z2, deg3, b2, wihT, whhT, bih, bhh, fcw, fcb):
    return pl.pallas_call(
        _lstm_body,
        grid=(NB,),
        in_specs=[
            pl.BlockSpec((T, NC, BN, HALF), lambda i: (0, 0, i, 0)),
            pl.BlockSpec((T, 1, 1, BN), lambda i: (0, i, 0, 0)),
            pl.BlockSpec((1, HID), lambda i: (0, 0)),
            pl.BlockSpec((HID, 4 * HID), lambda i: (0, 0)),
            pl.BlockSpec((HID, 4 * HID), lambda i: (0, 0)),
            pl.BlockSpec((1, 4 * HID), lambda i: (0, 0)),
            pl.BlockSpec((1, 4 * HID), lambda i: (0, 0)),
            pl.BlockSpec((HID, OUT_DIM), lambda i: (0, 0)),
            pl.BlockSpec((1, OUT_DIM), lambda i: (0, 0)),
        ],
        out_specs=pl.BlockSpec((BN, OUT_DIM), lambda i: (i, 0)),
        out_shape=jax.ShapeDtypeStruct((N, OUT_DIM), jnp.float32),
    )(z2, deg3, b2, wihT, whhT, bih, bhh, fcw, fcb)


def _edge_arrays(edge_indices):
    ei = edge_indices.astype(jnp.int32)
    pad = E_PAD - E
    src = jnp.pad(ei[:, 0], ((0, 0), (0, pad)))
    dst = jnp.pad(ei[:, 1], ((0, 0), (0, pad)), constant_values=N)
    src_r = src.reshape(T, NS, NG, 8, CHUNK)
    dst_r = dst.reshape(T, NS, NG, 8, CHUNK)
    return src_r, dst_r


@jax.jit
def _deg_run(edge_indices):
    _, dst_r = _edge_arrays(edge_indices)
    ones = jnp.ones((ZPT, HALF), jnp.float32)
    degp1 = _deg_kernel(dst_r, ones)
    return degp1[:, :, 0].reshape(T, NB, 1, BN)


@jax.jit
def _run(deg3, x, edge_indices, gcn1_W, gcn1_b, gcn2_W, gcn2_b, W_ih, W_hh,
         b_ih, b_hh, fc_W, fc_b):
    src_r, dst_r = _edge_arrays(edge_indices)

    xr = jnp.transpose(x[0], (1, 0, 2))
    y1 = _tc1_call(xr, deg3, gcn1_W)
    z1 = _edge_kernel(y1, src_r, dst_r)
    y2 = _tc2_call(z1, deg3, gcn1_b.reshape(1, HID), gcn2_W)
    z2 = _edge_kernel(y2, src_r, dst_r)
    out = _lstm_call(z2, deg3, gcn2_b.reshape(1, HID), W_ih.T, W_hh.T,
                     b_ih.reshape(1, 4 * HID), b_hh.reshape(1, 4 * HID),
                     fc_W, fc_b.reshape(1, OUT_DIM))
    return out.reshape(1, N, OUT_DIM)


def kernel(x, edge_indices, gcn1_W, gcn1_b, gcn2_W, gcn2_b, W_ih, W_hh,
           b_ih, b_hh, fc_W, fc_b):
    deg3 = _deg_run(edge_indices)
    return _run(deg3, x, edge_indices, gcn1_W, gcn1_b, gcn2_W, gcn2_b,
                W_ih, W_hh, b_ih, b_hh, fc_W, fc_b)


# speedup vs baseline: 6.2735x; 1.0176x over previous
import functools

import jax
import jax.numpy as jnp
from jax import lax
from jax.experimental import pallas as pl
from jax.experimental.pallas import tpu as pltpu
from jax.experimental.pallas import tpu_sc as plsc

N = 10000
T = 10
D_IN = 128
HID = 256
OUT_DIM = 128
HALF = 128
E = 160000

NC = 2
NS = 16
CHUNK = 128
NCHUNK = 80
EPT = NCHUNK * CHUNK
E_PAD = EPT * NS
NROW = 10240
ZPT = NROW // NS
RPT = 624
TAIL0 = RPT * NS
TAIL = N - TAIL0

SROW = 128
NG = 10
GROW = 64
NSTG = 10
NSTR = 16

BN = 1000
NB = N // BN

T_PER_SC = T // NC

_mesh = plsc.VectorSubcoreMesh(
    core_axis_name="c", subcore_axis_name="s", num_cores=NC, num_subcores=NS)


@functools.partial(
    pl.kernel,
    out_type=jax.ShapeDtypeStruct((T, N, HALF), jnp.float32),
    mesh=_mesh,
    scratch_types=[
        pltpu.VMEM_SHARED((NROW, HALF), jnp.float32),
        pltpu.VMEM((NG, 8, CHUNK), jnp.int32),
        pltpu.VMEM((SROW, HALF), jnp.float32),
        pltpu.SemaphoreType.DMA,
    ],
)
def _deg_kernel(dst_hbm, ones_hbm, out_hbm, acc, dst_v, ones_v, sem):
    c = lax.axis_index("c")
    s = lax.axis_index("s")
    pltpu.sync_copy(ones_hbm.at[pl.ds(0, SROW)], ones_v)

    def per_t(i, carry):
        t = c * T_PER_SC + i
        pltpu.sync_copy(ones_hbm, acc.at[pl.ds(s * ZPT, ZPT)])
        pltpu.sync_copy(dst_hbm.at[t, s], dst_v)
        plsc.subcore_barrier()

        def fire(g, carry2):
            for r in range(8):
                pltpu.async_copy(ones_v, acc.at[dst_v.at[g, r]], sem,
                                 add=True)
            return carry2

        lax.fori_loop(0, NG, fire, 0)

        def drain(g, carry2):
            for r in range(8):
                pltpu.make_async_copy(
                    ones_v, acc.at[dst_v.at[g, r]], sem).wait()
            return carry2

        lax.fori_loop(0, NG, drain, 0)
        plsc.subcore_barrier()
        pltpu.sync_copy(acc.at[pl.ds(s * RPT, RPT)],
                        out_hbm.at[t, pl.ds(s * RPT, RPT)])

        @pl.when(s == NS - 1)
        def _():
            pltpu.sync_copy(acc.at[pl.ds(TAIL0, TAIL)],
                            out_hbm.at[t, pl.ds(TAIL0, TAIL)])

        plsc.subcore_barrier()
        return carry

    lax.fori_loop(0, T_PER_SC, per_t, 0)


@functools.partial(
    pl.kernel,
    out_type=jax.ShapeDtypeStruct((T, NC, N, HALF), jnp.float32),
    mesh=_mesh,
    scratch_types=[
        pltpu.VMEM_SHARED((NROW, HALF), jnp.float32),
        pltpu.VMEM((NSTR, GROW), jnp.int32),
        pltpu.VMEM((NSTR, GROW), jnp.int32),
        pltpu.VMEM((NSTR, GROW), jnp.int32),
        pltpu.VMEM((NSTR, GROW), jnp.int32),
        pltpu.VMEM((GROW, HALF), jnp.float32),
        pltpu.VMEM((GROW, HALF), jnp.float32),
        pltpu.VMEM((GROW, HALF), jnp.float32),
        pltpu.VMEM((GROW, HALF), jnp.float32),
        pltpu.SemaphoreType.DMA,
        pltpu.SemaphoreType.DMA,
        pltpu.SemaphoreType.DMA,
        pltpu.SemaphoreType.DMA,
        pltpu.SemaphoreType.DMA,
        pltpu.SemaphoreType.DMA,
        pltpu.SemaphoreType.DMA,
        pltpu.SemaphoreType.DMA,
        pltpu.SemaphoreType.DMA,
        pltpu.SemaphoreType.DMA,
    ],
)
def _edge_kernel(y_hbm, src_hbm, dst_hbm, out_hbm, acc,
                 sv0, dv0, sv1, dv1, buf0, buf1, buf2, buf3,
                 gs0, gs1, gs2, gs3, ss0, ss1, ss2, ss3, is0, is1):
    c = lax.axis_index("c")
    s = lax.axis_index("s")
    BUFS = (buf0, buf1, buf2, buf3)
    GS = (gs0, gs1, gs2, gs3)
    SS = (ss0, ss1, ss2, ss3)

    def per_t(t, carry):
        yt = y_hbm.at[t, c]
        hbm64 = y_hbm.at[t, c, pl.ds(0, GROW)]

        def emit_stage(sv, dv, svn, isemn, reload_fn):
            for k in range(NSTR):
                b = k % 4
                pltpu.make_async_copy(yt.at[sv.at[k]], BUFS[b], GS[b]).wait()
                pltpu.async_copy(BUFS[b], acc.at[dv.at[k]], SS[b], add=True)
                ob = (k + 2) % 4
                pltpu.make_async_copy(hbm64, BUFS[ob], SS[ob]).wait()
                if k == 2 and reload_fn is not None:
                    reload_fn()
                if k < NSTR - 2:
                    pltpu.async_copy(yt.at[sv.at[k + 2]], BUFS[ob], GS[ob])
                else:
                    if k == NSTR - 2:
                        pltpu.make_async_copy(src_hbm.at[t, s, 0], svn,
                                              isemn).wait()
                        pltpu.make_async_copy(src_hbm.at[t, s, 0],
                                              dv0 if svn is sv0 else dv1,
                                              isemn).wait()
                    pltpu.async_copy(yt.at[svn.at[k - (NSTR - 2)]], BUFS[ob],
                                     GS[ob])

        pltpu.sync_copy(y_hbm.at[t, c, pl.ds(s * RPT, RPT)],
                        acc.at[pl.ds(s * RPT, RPT)])

        @pl.when(s == NS - 1)
        def _():
            pltpu.sync_copy(y_hbm.at[t, c, pl.ds(TAIL0, TAIL)],
                            acc.at[pl.ds(TAIL0, TAIL)])

        pltpu.sync_copy(src_hbm.at[t, s, 0], sv0)
        pltpu.sync_copy(dst_hbm.at[t, s, 0], dv0)
        plsc.subcore_barrier()

        pltpu.async_copy(yt.at[sv0.at[0]], buf0, gs0)
        pltpu.async_copy(yt.at[sv0.at[1]], buf1, gs1)
        pltpu.async_copy(hbm64, buf2, ss2)
        pltpu.async_copy(hbm64, buf3, ss3)

        def body(sp, carry2):
            sb = 2 * sp + 1

            def load_set1():
                pltpu.async_copy(src_hbm.at[t, s, sb], sv1, is1)
                pltpu.async_copy(dst_hbm.at[t, s, sb], dv1, is1)

            def load_set0():
                sn = jnp.minimum(2 * sp + 2, NSTG - 1)
                pltpu.async_copy(src_hbm.at[t, s, sn], sv0, is0)
                pltpu.async_copy(dst_hbm.at[t, s, sn], dv0, is0)

            emit_stage(sv0, dv0, sv1, is1, load_set1)
            emit_stage(sv1, dv1, sv0, is0, load_set0)
            return carry2

        lax.fori_loop(0, NSTG // 2, body, 0)
        pltpu.make_async_copy(hbm64, buf0, gs0).wait()
        pltpu.make_async_copy(hbm64, buf1, gs1).wait()
        pltpu.make_async_copy(hbm64, buf2, ss2).wait()
        pltpu.make_async_copy(hbm64, buf3, ss3).wait()
        plsc.subcore_barrier()
        pltpu.sync_copy(acc.at[pl.ds(s * RPT, RPT)],
                        out_hbm.at[t, c, pl.ds(s * RPT, RPT)])

        @pl.when(s == NS - 1)
        def _():
            pltpu.sync_copy(acc.at[pl.ds(TAIL0, TAIL)],
                            out_hbm.at[t, c, pl.ds(TAIL0, TAIL)])

        plsc.subcore_barrier()
        return carry

    lax.fori_loop(0, T, per_t, 0)


def _tc1_body(x_ref, deg_ref, w_ref, o_ref):
    xt = x_ref[0]
    y = jnp.dot(xt, w_ref[...], preferred_element_type=jnp.float32)
    dinv = lax.rsqrt(deg_ref[0, 0, 0])
    y = y * dinv[:, None]
    o_ref[0, 0] = y[:, :HALF]
    o_ref[0, 1] = y[:, HALF:]


def _tc2_body(z_ref, deg_ref, b1_ref, w2_ref, o_ref):
    z = jnp.concatenate([z_ref[0, 0], z_ref[0, 1]], axis=1)
    dinv = lax.rsqrt(deg_ref[0, 0, 0])
    h = jnp.maximum(z * dinv[:, None] + b1_ref[...], 0.0)
    y = jnp.dot(h, w2_ref[...], preferred_element_type=jnp.float32)
    y = y * dinv[:, None]
    o_ref[0, 0] = y[:, :HALF]
    o_ref[0, 1] = y[:, HALF:]


def _lstm_body(z_ref, deg_ref, b2_ref, wih_ref, whh_ref, bih_ref, bhh_ref,
               fcw_ref, fcb_ref, o_ref):
    h = jnp.zeros((BN, HID), jnp.float32)
    cc = jnp.zeros((BN, HID), jnp.float32)
    bg = bih_ref[...] + bhh_ref[...]
    for t in range(T):
        z = jnp.concatenate([z_ref[t, 0], z_ref[t, 1]], axis=1)
        dinv = lax.rsqrt(deg_ref[t, 0, 0])
        xt = z * dinv[:, None] + b2_ref[...]
        g = (jnp.dot(xt, wih_ref[...], preferred_element_type=jnp.float32)
             + jnp.dot(h, whh_ref[...], preferred_element_type=jnp.float32)
             + bg)
        gi = jax.nn.sigmoid(g[:, :HID])
        gf = jax.nn.sigmoid(g[:, HID:2 * HID])
        gg = jnp.tanh(g[:, 2 * HID:3 * HID])
        go = jax.nn.sigmoid(g[:, 3 * HID:])
        cc = gf * cc + gi * gg
        h = go * jnp.tanh(cc)
    o_ref[...] = (jnp.dot(h, fcw_ref[...], preferred_element_type=jnp.float32)
                  + fcb_ref[...])


def _tc1_call(xr, deg3, w1):
    return pl.pallas_call(
        _tc1_body,
        grid=(T, NB),
        in_specs=[
            pl.BlockSpec((1, BN, D_IN), lambda t, i: (t, i, 0)),
            pl.BlockSpec((1, 1, 1, BN), lambda t, i: (t, i, 0, 0)),
            pl.BlockSpec((D_IN, HID), lambda t, i: (0, 0)),
        ],
        out_specs=pl.BlockSpec((1, NC, BN, HALF), lambda t, i: (t, 0, i, 0)),
        out_shape=jax.ShapeDtypeStruct((T, NC, N, HALF), jnp.float32),
    )(xr, deg3, w1)


def _tc2_call(z1, deg3, b1, w2):
    return pl.pallas_call(
        _tc2_body,
        grid=(T, NB),
        in_specs=[
            pl.BlockSpec((1, NC, BN, HALF), lambda t, i: (t, 0, i, 0)),
            pl.BlockSpec((1, 1, 1, BN), lambda t, i: (t, i, 0, 0)),
            pl.BlockSpec((1, HID), lambda t, i: (0, 0)),
            pl.BlockSpec((HID, HID), lambda t, i: (0, 0)),
        ],
        out_specs=pl.BlockSpec((1, NC, BN, HALF), lambda t, i: (t, 0, i, 0)),
        out_shape=jax.ShapeDtypeStruct((T, NC, N, HALF), jnp.float32),
    )(z1, deg3, b1, w2)


def _lstm_call(z2, deg3, b2, wihT, whhT, bih, bhh, fcw, fcb):
    return pl.pallas_call(
        _lstm_body,
        grid=(NB,),
        in_specs=[
            pl.BlockSpec((T, NC, BN, HALF), lambda i: (0, 0, i, 0)),
            pl.BlockSpec((T, 1, 1, BN), lambda i: (0, i, 0, 0)),
            pl.BlockSpec((1, HID), lambda i: (0, 0)),
            pl.BlockSpec((HID, 4 * HID), lambda i: (0, 0)),
            pl.BlockSpec((HID, 4 * HID), lambda i: (0, 0)),
            pl.BlockSpec((1, 4 * HID), lambda i: (0, 0)),
            pl.BlockSpec((1, 4 * HID), lambda i: (0, 0)),
            pl.BlockSpec((HID, OUT_DIM), lambda i: (0, 0)),
            pl.BlockSpec((1, OUT_DIM), lambda i: (0, 0)),
        ],
        out_specs=pl.BlockSpec((BN, OUT_DIM), lambda i: (i, 0)),
        out_shape=jax.ShapeDtypeStruct((N, OUT_DIM), jnp.float32),
    )(z2, deg3, b2, wihT, whhT, bih, bhh, fcw, fcb)


def _edge_arrays(edge_indices):
    ei = edge_indices.astype(jnp.int32)
    pad = E_PAD - E
    src = jnp.pad(ei[:, 0], ((0, 0), (0, pad)))
    dst = jnp.pad(ei[:, 1], ((0, 0), (0, pad)), constant_values=N)
    src_r = src.reshape(T, NS, NSTG, NSTR, GROW)
    dst_r = dst.reshape(T, NS, NSTG, NSTR, GROW)
    return src_r, dst_r


@jax.jit
def _deg_run(edge_indices):
    ei = edge_indices.astype(jnp.int32)
    pad = E_PAD - E
    dst = jnp.pad(ei[:, 1], ((0, 0), (0, pad)), constant_values=N)
    dst_r = dst.reshape(T, NS, NG, 8, CHUNK)
    ones = jnp.ones((ZPT, HALF), jnp.float32)
    degp1 = _deg_kernel(dst_r, ones)
    return degp1[:, :, 0].reshape(T, NB, 1, BN)


@jax.jit
def _run(deg3, x, edge_indices, gcn1_W, gcn1_b, gcn2_W, gcn2_b, W_ih, W_hh,
         b_ih, b_hh, fc_W, fc_b):
    src_r, dst_r = _edge_arrays(edge_indices)

    xr = jnp.transpose(x[0], (1, 0, 2))
    y1 = _tc1_call(xr, deg3, gcn1_W)
    z1 = _edge_kernel(y1, src_r, dst_r)
    y2 = _tc2_call(z1, deg3, gcn1_b.reshape(1, HID), gcn2_W)
    z2 = _edge_kernel(y2, src_r, dst_r)
    out = _lstm_call(z2, deg3, gcn2_b.reshape(1, HID), W_ih.T, W_hh.T,
                     b_ih.reshape(1, 4 * HID), b_hh.reshape(1, 4 * HID),
                     fc_W, fc_b.reshape(1, OUT_DIM))
    return out.reshape(1, N, OUT_DIM)


def kernel(x, edge_indices, gcn1_W, gcn1_b, gcn2_W, gcn2_b, W_ih, W_hh,
           b_ih, b_hh, fc_W, fc_b):
    deg3 = _deg_run(edge_indices)
    return _run(deg3, x, edge_indices, gcn1_W, gcn1_b, gcn2_W, gcn2_b,
                W_ih, W_hh, b_ih, b_hh, fc_W, fc_b)


# speedup vs baseline: 13.2337x; 2.1094x over previous
import functools

import jax
import jax.numpy as jnp
from jax import lax
from jax.experimental import pallas as pl
from jax.experimental.pallas import tpu as pltpu
from jax.experimental.pallas import tpu_sc as plsc

N = 10000
T = 10
D_IN = 128
HID = 256
OUT_DIM = 128
HALF = 128
E = 160000

NC = 2
NS = 16
CHUNK = 128
NCHUNK = 80
EPT = NCHUNK * CHUNK
E_PAD = EPT * NS
NROW = 10240
ZPT = NROW // NS
RPT = 624
TAIL0 = RPT * NS
TAIL = N - TAIL0

SROW = 128
NG = 10
GROW = 64
NSTG = 10
NSTR = 16

BN = 1000
NB = N // BN

T_PER_SC = T // NC

_mesh = plsc.VectorSubcoreMesh(
    core_axis_name="c", subcore_axis_name="s", num_cores=NC, num_subcores=NS)


@functools.partial(
    pl.kernel,
    out_type=jax.ShapeDtypeStruct((T, N, HALF), jnp.float32),
    mesh=_mesh,
    scratch_types=[
        pltpu.VMEM_SHARED((NROW, HALF), jnp.float32),
        pltpu.VMEM((NG, 8, CHUNK), jnp.int32),
        pltpu.VMEM((SROW, HALF), jnp.float32),
        pltpu.SemaphoreType.DMA,
    ],
)
def _deg_kernel(dst_hbm, ones_hbm, out_hbm, acc, dst_v, ones_v, sem):
    c = lax.axis_index("c")
    s = lax.axis_index("s")
    pltpu.sync_copy(ones_hbm.at[pl.ds(0, SROW)], ones_v)

    def per_t(i, carry):
        t = c * T_PER_SC + i
        pltpu.sync_copy(ones_hbm, acc.at[pl.ds(s * ZPT, ZPT)])
        pltpu.sync_copy(dst_hbm.at[t, s], dst_v)
        plsc.subcore_barrier()

        def fire(g, carry2):
            for r in range(8):
                pltpu.async_copy(ones_v, acc.at[dst_v.at[g, r]], sem,
                                 add=True)
            return carry2

        lax.fori_loop(0, NG, fire, 0)

        def drain(g, carry2):
            for r in range(8):
                pltpu.make_async_copy(
                    ones_v, acc.at[dst_v.at[g, r]], sem).wait()
            return carry2

        lax.fori_loop(0, NG, drain, 0)
        plsc.subcore_barrier()
        pltpu.sync_copy(acc.at[pl.ds(s * RPT, RPT)],
                        out_hbm.at[t, pl.ds(s * RPT, RPT)])

        @pl.when(s == NS - 1)
        def _():
            pltpu.sync_copy(acc.at[pl.ds(TAIL0, TAIL)],
                            out_hbm.at[t, pl.ds(TAIL0, TAIL)])

        plsc.subcore_barrier()
        return carry

    lax.fori_loop(0, T_PER_SC, per_t, 0)


@functools.partial(
    pl.kernel,
    out_type=jax.ShapeDtypeStruct((T, NC, N, HALF), jnp.float32),
    mesh=_mesh,
    scratch_types=[
        pltpu.VMEM_SHARED((NROW, HALF), jnp.float32),
        pltpu.VMEM((NSTR, GROW), jnp.int32),
        pltpu.VMEM((NSTR, GROW), jnp.int32),
        pltpu.VMEM((NSTR, GROW), jnp.int32),
        pltpu.VMEM((NSTR, GROW), jnp.int32),
        pltpu.VMEM((GROW, HALF), jnp.float32),
        pltpu.VMEM((GROW, HALF), jnp.float32),
        pltpu.VMEM((GROW, HALF), jnp.float32),
        pltpu.VMEM((GROW, HALF), jnp.float32),
        pltpu.SemaphoreType.DMA,
        pltpu.SemaphoreType.DMA,
        pltpu.SemaphoreType.DMA,
        pltpu.SemaphoreType.DMA,
        pltpu.SemaphoreType.DMA,
        pltpu.SemaphoreType.DMA,
        pltpu.SemaphoreType.DMA,
        pltpu.SemaphoreType.DMA,
        pltpu.SemaphoreType.DMA,
        pltpu.SemaphoreType.DMA,
    ],
)
def _edge_kernel(y_hbm, src_hbm, dst_hbm, out_hbm, acc,
                 sv0, dv0, sv1, dv1, buf0, buf1, buf2, buf3,
                 gs0, gs1, gs2, gs3, ss0, ss1, ss2, ss3, is0, is1):
    c = lax.axis_index("c")
    s = lax.axis_index("s")
    BUFS = (buf0, buf1, buf2, buf3)
    GS = (gs0, gs1, gs2, gs3)
    SS = (ss0, ss1, ss2, ss3)

    def per_t(t, carry):
        yt = y_hbm.at[t, c]
        hbm64 = y_hbm.at[t, c, pl.ds(0, GROW)]

        def emit_stage(sv, dv, svn, isemn, reload_fn):
            for k in range(NSTR):
                b = k % 4
                pltpu.make_async_copy(yt.at[sv.at[k]], BUFS[b], GS[b]).wait()
                pltpu.async_copy(BUFS[b], acc.at[dv.at[k]], SS[b], add=True)
                ob = (k + 2) % 4
                pltpu.make_async_copy(hbm64, BUFS[ob], SS[ob]).wait()
                if k == 2 and reload_fn is not None:
                    reload_fn()
                if k < NSTR - 2:
                    pltpu.async_copy(yt.at[sv.at[k + 2]], BUFS[ob], GS[ob])
                else:
                    if k == NSTR - 2:
                        pltpu.make_async_copy(src_hbm.at[t, s, 0], svn,
                                              isemn).wait()
                        pltpu.make_async_copy(src_hbm.at[t, s, 0],
                                              dv0 if svn is sv0 else dv1,
                                              isemn).wait()
                    pltpu.async_copy(yt.at[svn.at[k - (NSTR - 2)]], BUFS[ob],
                                     GS[ob])

        pltpu.sync_copy(y_hbm.at[t, c, pl.ds(s * RPT, RPT)],
                        acc.at[pl.ds(s * RPT, RPT)])

        @pl.when(s == NS - 1)
        def _():
            pltpu.sync_copy(y_hbm.at[t, c, pl.ds(TAIL0, TAIL)],
                            acc.at[pl.ds(TAIL0, TAIL)])

        pltpu.sync_copy(src_hbm.at[t, s, 0], sv0)
        pltpu.sync_copy(dst_hbm.at[t, s, 0], dv0)
        plsc.subcore_barrier()

        pltpu.async_copy(yt.at[sv0.at[0]], buf0, gs0)
        pltpu.async_copy(yt.at[sv0.at[1]], buf1, gs1)
        pltpu.async_copy(hbm64, buf2, ss2)
        pltpu.async_copy(hbm64, buf3, ss3)

        def body(sp, carry2):
            sb = 2 * sp + 1

            def load_set1():
                pltpu.async_copy(src_hbm.at[t, s, sb], sv1, is1)
                pltpu.async_copy(dst_hbm.at[t, s, sb], dv1, is1)

            def load_set0():
                sn = jnp.minimum(2 * sp + 2, NSTG - 1)
                pltpu.async_copy(src_hbm.at[t, s, sn], sv0, is0)
                pltpu.async_copy(dst_hbm.at[t, s, sn], dv0, is0)

            emit_stage(sv0, dv0, sv1, is1, load_set1)
            emit_stage(sv1, dv1, sv0, is0, load_set0)
            return carry2

        lax.fori_loop(0, NSTG // 2, body, 0)
        pltpu.make_async_copy(hbm64, buf0, gs0).wait()
        pltpu.make_async_copy(hbm64, buf1, gs1).wait()
        pltpu.make_async_copy(hbm64, buf2, ss2).wait()
        pltpu.make_async_copy(hbm64, buf3, ss3).wait()
        plsc.subcore_barrier()
        pltpu.sync_copy(acc.at[pl.ds(s * RPT, RPT)],
                        out_hbm.at[t, c, pl.ds(s * RPT, RPT)])

        @pl.when(s == NS - 1)
        def _():
            pltpu.sync_copy(acc.at[pl.ds(TAIL0, TAIL)],
                            out_hbm.at[t, c, pl.ds(TAIL0, TAIL)])

        plsc.subcore_barrier()
        return carry

    lax.fori_loop(0, T, per_t, 0)


def _tc1_body(x_ref, deg_ref, w_ref, o_ref):
    xt = x_ref[0]
    y = jnp.dot(xt, w_ref[...], preferred_element_type=jnp.float32)
    dinv = lax.rsqrt(deg_ref[0, 0, 0])
    y = y * dinv[:, None]
    o_ref[0, 0] = y[:, :HALF]
    o_ref[0, 1] = y[:, HALF:]


def _tc2_body(z_ref, deg_ref, b1_ref, w2_ref, o_ref):
    z = jnp.concatenate([z_ref[0, 0], z_ref[0, 1]], axis=1)
    dinv = lax.rsqrt(deg_ref[0, 0, 0])
    h = jnp.maximum(z * dinv[:, None] + b1_ref[...], 0.0)
    y = jnp.dot(h, w2_ref[...], preferred_element_type=jnp.float32)
    y = y * dinv[:, None]
    o_ref[0, 0] = y[:, :HALF]
    o_ref[0, 1] = y[:, HALF:]


def _lstm_body(z_ref, deg_ref, b2_ref, wih_ref, whh_ref, bih_ref, bhh_ref,
               fcw_ref, fcb_ref, o_ref):
    h = jnp.zeros((BN, HID), jnp.float32)
    cc = jnp.zeros((BN, HID), jnp.float32)
    bg = bih_ref[...] + bhh_ref[...]
    for t in range(T):
        z = jnp.concatenate([z_ref[t, 0], z_ref[t, 1]], axis=1)
        dinv = lax.rsqrt(deg_ref[t, 0, 0])
        xt = z * dinv[:, None] + b2_ref[...]
        g = (jnp.dot(xt, wih_ref[...], preferred_element_type=jnp.float32)
             + jnp.dot(h, whh_ref[...], preferred_element_type=jnp.float32)
             + bg)
        gi = jax.nn.sigmoid(g[:, :HID])
        gf = jax.nn.sigmoid(g[:, HID:2 * HID])
        gg = jnp.tanh(g[:, 2 * HID:3 * HID])
        go = jax.nn.sigmoid(g[:, 3 * HID:])
        cc = gf * cc + gi * gg
        h = go * jnp.tanh(cc)
    o_ref[...] = (jnp.dot(h, fcw_ref[...], preferred_element_type=jnp.float32)
                  + fcb_ref[...])


def _tc1_call(xr, deg3, w1):
    return pl.pallas_call(
        _tc1_body,
        grid=(T, NB),
        in_specs=[
            pl.BlockSpec((1, BN, D_IN), lambda t, i: (t, i, 0)),
            pl.BlockSpec((1, 1, 1, BN), lambda t, i: (t, i, 0, 0)),
            pl.BlockSpec((D_IN, HID), lambda t, i: (0, 0)),
        ],
        out_specs=pl.BlockSpec((1, NC, BN, HALF), lambda t, i: (t, 0, i, 0)),
        out_shape=jax.ShapeDtypeStruct((T, NC, N, HALF), jnp.float32),
    )(xr, deg3, w1)


def _tc2_call(z1, deg3, b1, w2):
    return pl.pallas_call(
        _tc2_body,
        grid=(T, NB),
        in_specs=[
            pl.BlockSpec((1, NC, BN, HALF), lambda t, i: (t, 0, i, 0)),
            pl.BlockSpec((1, 1, 1, BN), lambda t, i: (t, i, 0, 0)),
            pl.BlockSpec((1, HID), lambda t, i: (0, 0)),
            pl.BlockSpec((HID, HID), lambda t, i: (0, 0)),
        ],
        out_specs=pl.BlockSpec((1, NC, BN, HALF), lambda t, i: (t, 0, i, 0)),
        out_shape=jax.ShapeDtypeStruct((T, NC, N, HALF), jnp.float32),
    )(z1, deg3, b1, w2)


def _lstm_call(z2, deg3, b2, wihT, whhT, bih, bhh, fcw, fcb):
    return pl.pallas_call(
        _lstm_body,
        grid=(NB,),
        in_specs=[
            pl.BlockSpec((T, NC, BN, HALF), lambda i: (0, 0, i, 0)),
            pl.BlockSpec((T, 1, 1, BN), lambda i: (0, i, 0, 0)),
            pl.BlockSpec((1, HID), lambda i: (0, 0)),
            pl.BlockSpec((HID, 4 * HID), lambda i: (0, 0)),
            pl.BlockSpec((HID, 4 * HID), lambda i: (0, 0)),
            pl.BlockSpec((1, 4 * HID), lambda i: (0, 0)),
            pl.BlockSpec((1, 4 * HID), lambda i: (0, 0)),
            pl.BlockSpec((HID, OUT_DIM), lambda i: (0, 0)),
            pl.BlockSpec((1, OUT_DIM), lambda i: (0, 0)),
        ],
        out_specs=pl.BlockSpec((BN, OUT_DIM), lambda i: (i, 0)),
        out_shape=jax.ShapeDtypeStruct((N, OUT_DIM), jnp.float32),
    )(z2, deg3, b2, wihT, whhT, bih, bhh, fcw, fcb)


def _edge_arrays(edge_indices):
    ei = edge_indices.astype(jnp.int32)
    pad = E_PAD - E
    spread_src = jnp.broadcast_to(jnp.arange(pad, dtype=jnp.int32) % N,
                                  (T, pad))
    spread_dst = jnp.broadcast_to(
        N + (jnp.arange(pad, dtype=jnp.int32) % (NROW - N)), (T, pad))
    src = jnp.concatenate([ei[:, 0], spread_src], axis=1)
    dst = jnp.concatenate([ei[:, 1], spread_dst], axis=1)
    src_r = src.reshape(T, NS, NSTG, NSTR, GROW)
    dst_r = dst.reshape(T, NS, NSTG, NSTR, GROW)
    return src_r, dst_r


@jax.jit
def _deg_run(edge_indices):
    ei = edge_indices.astype(jnp.int32)
    pad = E_PAD - E
    spread_dst = jnp.broadcast_to(
        N + (jnp.arange(pad, dtype=jnp.int32) % (NROW - N)), (T, pad))
    dst = jnp.concatenate([ei[:, 1], spread_dst], axis=1)
    dst_r = dst.reshape(T, NS, NG, 8, CHUNK)
    ones = jnp.ones((ZPT, HALF), jnp.float32)
    degp1 = _deg_kernel(dst_r, ones)
    return degp1[:, :, 0].reshape(T, NB, 1, BN)


@jax.jit
def _run(deg3, x, edge_indices, gcn1_W, gcn1_b, gcn2_W, gcn2_b, W_ih, W_hh,
         b_ih, b_hh, fc_W, fc_b):
    src_r, dst_r = _edge_arrays(edge_indices)

    xr = jnp.transpose(x[0], (1, 0, 2))
    y1 = _tc1_call(xr, deg3, gcn1_W)
    z1 = _edge_kernel(y1, src_r, dst_r)
    y2 = _tc2_call(z1, deg3, gcn1_b.reshape(1, HID), gcn2_W)
    z2 = _edge_kernel(y2, src_r, dst_r)
    out = _lstm_call(z2, deg3, gcn2_b.reshape(1, HID), W_ih.T, W_hh.T,
                     b_ih.reshape(1, 4 * HID), b_hh.reshape(1, 4 * HID),
                     fc_W, fc_b.reshape(1, OUT_DIM))
    return out.reshape(1, N, OUT_DIM)


def kernel(x, edge_indices, gcn1_W, gcn1_b, gcn2_W, gcn2_b, W_ih, W_hh,
           b_ih, b_hh, fc_W, fc_b):
    deg3 = _deg_run(edge_indices)
    return _run(deg3, x, edge_indices, gcn1_W, gcn1_b, gcn2_W, gcn2_b,
                W_ih, W_hh, b_ih, b_hh, fc_W, fc_b)
